# Initial kernel scaffold; baseline (speedup 1.0000x reference)
import functools

import jax
import jax.numpy as jnp
from jax import lax
from jax.experimental import pallas as pl
from jax.experimental.pallas import tpu as pltpu
from jax.experimental.pallas import tpu_sc as plsc

_NC = 2
_NS = 16
_NW = _NC * _NS
_C = 128


def _sc_mesh():
    return plsc.VectorSubcoreMesh(core_axis_name="c", subcore_axis_name="s")


def _make_deg_kernel(V, T):
    rpt = V // _NS

    @functools.partial(
        pl.kernel,
        out_type=jax.ShapeDtypeStruct((_NC, V, 16), jnp.float32),
        mesh=_sc_mesh(),
        scratch_types=[
            pltpu.VMEM((T, _C), jnp.int32),
            pltpu.VMEM((_C, 16), jnp.float32),
            pltpu.VMEM_SHARED((V, 16), jnp.float32),
            pltpu.SemaphoreType.DMA,
        ],
    )
    def deg_kernel(ones_hbm, zeros_hbm, dst_hbm, out_hbm, dst_v, rows_v, acc_sh, sem):
        c = lax.axis_index("c")
        s = lax.axis_index("s")
        w = s * _NC + c
        pltpu.sync_copy(dst_hbm.at[w], dst_v)
        pltpu.sync_copy(zeros_hbm, rows_v)
        for i in range(rpt // _C):
            pltpu.sync_copy(rows_v, acc_sh.at[pl.ds(s * rpt + i * _C, _C)])
        plsc.subcore_barrier()
        pltpu.sync_copy(ones_hbm, rows_v)

        def body(j, carry):
            pltpu.sync_copy(rows_v, acc_sh.at[dst_v.at[j]], add=True)
            return carry

        lax.fori_loop(0, T, body, 0)
        plsc.subcore_barrier()
        for i in range(rpt // _C):
            pltpu.sync_copy(acc_sh.at[pl.ds(s * rpt + i * _C, _C)], rows_v)
            pltpu.sync_copy(rows_v, out_hbm.at[c, pl.ds(s * rpt + i * _C, _C)])

    return deg_kernel


def _make_prop_kernel(V, D, T):
    rpt = V // _NS

    @functools.partial(
        pl.kernel,
        out_type=jax.ShapeDtypeStruct((_NC, V, D), jnp.float32),
        mesh=_sc_mesh(),
        scratch_types=[
            pltpu.VMEM((T, _C), jnp.int32),
            pltpu.VMEM((T, _C), jnp.int32),
            pltpu.VMEM((_C, D), jnp.float32),
            pltpu.VMEM_SHARED((V, D), jnp.float32),
            pltpu.SemaphoreType.DMA,
        ],
    )
    def prop_kernel(g_hbm, zeros_hbm, src_hbm, dst_hbm, out_hbm,
                    src_v, dst_v, rows_v, acc_sh, sem):
        c = lax.axis_index("c")
        s = lax.axis_index("s")
        w = s * _NC + c
        pltpu.sync_copy(src_hbm.at[w], src_v)
        pltpu.sync_copy(dst_hbm.at[w], dst_v)
        pltpu.sync_copy(zeros_hbm, rows_v)
        for i in range(rpt // _C):
            pltpu.sync_copy(rows_v, acc_sh.at[pl.ds(s * rpt + i * _C, _C)])
        plsc.subcore_barrier()

        def body(j, carry):
            pltpu.async_copy(g_hbm.at[src_v.at[j]], rows_v, sem).wait()
            pltpu.sync_copy(rows_v, acc_sh.at[dst_v.at[j]], add=True)
            return carry

        lax.fori_loop(0, T, body, 0)
        plsc.subcore_barrier()
        for i in range(rpt // _C):
            pltpu.sync_copy(acc_sh.at[pl.ds(s * rpt + i * _C, _C)], rows_v)
            pltpu.sync_copy(rows_v, out_hbm.at[c, pl.ds(s * rpt + i * _C, _C)])

    return prop_kernel


def _dinv_from(deg_ref):
    cnt = deg_ref[0, :, 0:1] + deg_ref[1, :, 0:1]
    return lax.rsqrt(cnt + 1.0)


def _t_first(x_ref, w_ref, deg_ref, o_ref):
    dinv = _dinv_from(deg_ref)
    o_ref[...] = dinv * jnp.dot(x_ref[...], w_ref[...],
                                preferred_element_type=jnp.float32)


def _t_mid(p_ref, g_ref, deg_ref, w_ref, b_ref, o_ref):
    dinv = _dinv_from(deg_ref)
    h = dinv * (p_ref[0] + p_ref[1] + g_ref[...]) + b_ref[...]
    h = jnp.maximum(h, 0.0)
    o_ref[...] = dinv * jnp.dot(h, w_ref[...],
                                preferred_element_type=jnp.float32)


def _t_last(p_ref, g_ref, deg_ref, b_ref, o_ref):
    dinv = _dinv_from(deg_ref)
    o_ref[...] = dinv * (p_ref[0] + p_ref[1] + g_ref[...]) + b_ref[...]


def _tc_call(body, grid, V, B, out_d, in_specs):
    return pl.pallas_call(
        body,
        grid=(grid,),
        in_specs=in_specs,
        out_specs=pl.BlockSpec((B, out_d), lambda i: (i, 0)),
        out_shape=jax.ShapeDtypeStruct((V, out_d), jnp.float32),
    )


def kernel(x, edge_index, W1, b1, W2, b2, W3, b3):
    N, DIN = x.shape
    DH = W1.shape[1]
    DOUT = W3.shape[1]
    V = ((N + 1 + 2047) // 2048) * 2048
    src = edge_index[0]
    dst = edge_index[1]
    E = src.shape[0]
    T = -(-E // (_NW * _C))
    EP = _NW * T * _C
    pad = EP - E
    src3 = jnp.concatenate([src, jnp.full((pad,), N, jnp.int32)]).reshape(_NW, T, _C)
    dst3 = jnp.concatenate([dst, jnp.full((pad,), N, jnp.int32)]).reshape(_NW, T, _C)

    x_p = jnp.pad(x, ((0, V - N), (0, 0)))
    W3p = jnp.pad(W3, ((0, 0), (0, 16 - DOUT)))
    b1r = b1.reshape(1, DH)
    b2r = b2.reshape(1, DH)
    b3r = jnp.pad(b3, (0, 16 - DOUT)).reshape(1, 16)

    ones16 = jnp.ones((_C, 16), jnp.float32)
    zeros16 = jnp.zeros((_C, 16), jnp.float32)
    zerosD = jnp.zeros((_C, DH), jnp.float32)

    deg_k = _make_deg_kernel(V, T)
    prop_k = _make_prop_kernel(V, DH, T)
    prop_k16 = _make_prop_kernel(V, 16, T)

    deg = deg_k(ones16, zeros16, dst3)

    B = 512
    G = V // B
    spec_w = pl.BlockSpec((DIN, DH), lambda i: (0, 0))
    spec_deg = pl.BlockSpec((2, B, 16), lambda i: (0, i, 0))
    spec_row = pl.BlockSpec((B, DH), lambda i: (i, 0))
    spec_row16 = pl.BlockSpec((B, 16), lambda i: (i, 0))
    spec_p = pl.BlockSpec((2, B, DH), lambda i: (0, i, 0))
    spec_p16 = pl.BlockSpec((2, B, 16), lambda i: (0, i, 0))
    spec_b = pl.BlockSpec((1, DH), lambda i: (0, 0))
    spec_b16 = pl.BlockSpec((1, 16), lambda i: (0, 0))

    g1 = _tc_call(_t_first, G, V, B, DH,
                  [spec_row, spec_w, spec_deg])(x_p, W1, deg)
    p1 = prop_k(g1, zerosD, src3, dst3)
    g2 = _tc_call(_t_mid, G, V, B, DH,
                  [spec_p, spec_row, spec_deg, spec_w, spec_b])(p1, g1, deg, W2, b1r)
    p2 = prop_k(g2, zerosD, src3, dst3)
    spec_w16 = pl.BlockSpec((DH, 16), lambda i: (0, 0))
    g3 = _tc_call(_t_mid, G, V, B, 16,
                  [spec_p, spec_row, spec_deg, spec_w16, spec_b])(p2, g2, deg, W3p, b2r)
    p3 = prop_k16(g3, zeros16, src3, dst3)
    out = _tc_call(_t_last, G, V, B, 16,
                   [spec_p16, spec_row16, spec_deg, spec_b16])(p3, g3, deg, b3r)
    return out[:N, :DOUT]


# speedup vs baseline: 13.6332x; 13.6332x over previous
import functools

import jax
import jax.numpy as jnp
from jax import lax
from jax.experimental import pallas as pl
from jax.experimental.pallas import tpu as pltpu
from jax.experimental.pallas import tpu_sc as plsc

_NC = 2
_NS = 16
_NW = _NC * _NS
_C = 128


def _sc_mesh():
    return plsc.VectorSubcoreMesh(core_axis_name="c", subcore_axis_name="s")


def _make_deg_kernel(V, T):
    rpt = V // _NS

    @functools.partial(
        pl.kernel,
        out_type=jax.ShapeDtypeStruct((_NC, V, 16), jnp.float32),
        mesh=_sc_mesh(),
        compiler_params=pltpu.CompilerParams(use_tc_tiling_on_sc=False),
        scratch_types=[
            pltpu.VMEM((T, _C), jnp.int32),
            pltpu.VMEM((_C, 16), jnp.float32),
            pltpu.VMEM_SHARED((V, 16), jnp.float32),
            pltpu.SemaphoreType.DMA,
        ],
    )
    def deg_kernel(ones_hbm, zeros_hbm, dst_hbm, out_hbm, dst_v, rows_v, acc_sh, sem):
        c = lax.axis_index("c")
        s = lax.axis_index("s")
        w = s * _NC + c
        pltpu.sync_copy(dst_hbm.at[w], dst_v)
        pltpu.sync_copy(zeros_hbm, rows_v)
        for i in range(rpt // _C):
            pltpu.sync_copy(rows_v, acc_sh.at[pl.ds(s * rpt + i * _C, _C)])
        plsc.subcore_barrier()
        pltpu.sync_copy(ones_hbm, rows_v)

        def body(j, carry):
            pltpu.sync_copy(rows_v, acc_sh.at[dst_v.at[j]], add=True)
            return carry

        lax.fori_loop(0, T, body, 0)
        plsc.subcore_barrier()
        for i in range(rpt // _C):
            pltpu.sync_copy(acc_sh.at[pl.ds(s * rpt + i * _C, _C)], rows_v)
            pltpu.sync_copy(rows_v, out_hbm.at[c, pl.ds(s * rpt + i * _C, _C)])

    return deg_kernel


def _make_prop_kernel(V, D, T):
    rpt = V // _NS

    @functools.partial(
        pl.kernel,
        out_type=jax.ShapeDtypeStruct((_NC, V, D), jnp.float32),
        mesh=_sc_mesh(),
        compiler_params=(None if D % 128 == 0
                         else pltpu.CompilerParams(use_tc_tiling_on_sc=False)),
        scratch_types=[
            pltpu.VMEM((T, _C), jnp.int32),
            pltpu.VMEM((T, _C), jnp.int32),
            pltpu.VMEM((_C, D), jnp.float32),
            pltpu.VMEM_SHARED((V, D), jnp.float32),
            pltpu.SemaphoreType.DMA,
        ],
    )
    def prop_kernel(g_hbm, zeros_hbm, src_hbm, dst_hbm, out_hbm,
                    src_v, dst_v, rows_v, acc_sh, sem):
        c = lax.axis_index("c")
        s = lax.axis_index("s")
        w = s * _NC + c
        pltpu.sync_copy(src_hbm.at[w], src_v)
        pltpu.sync_copy(dst_hbm.at[w], dst_v)
        pltpu.sync_copy(zeros_hbm, rows_v)
        for i in range(rpt // _C):
            pltpu.sync_copy(rows_v, acc_sh.at[pl.ds(s * rpt + i * _C, _C)])
        plsc.subcore_barrier()

        def body(j, carry):
            pltpu.async_copy(g_hbm.at[src_v.at[j]], rows_v, sem).wait()
            pltpu.sync_copy(rows_v, acc_sh.at[dst_v.at[j]], add=True)
            return carry

        lax.fori_loop(0, T, body, 0)
        plsc.subcore_barrier()
        for i in range(rpt // _C):
            pltpu.sync_copy(acc_sh.at[pl.ds(s * rpt + i * _C, _C)], rows_v)
            pltpu.sync_copy(rows_v, out_hbm.at[c, pl.ds(s * rpt + i * _C, _C)])

    return prop_kernel


def _dinv_from(deg_ref):
    cnt = deg_ref[0, :, 0:1] + deg_ref[1, :, 0:1]
    return lax.rsqrt(cnt + 1.0)


def _t_first(x_ref, w_ref, deg_ref, o_ref):
    dinv = _dinv_from(deg_ref)
    o_ref[...] = dinv * jnp.dot(x_ref[...], w_ref[...],
                                preferred_element_type=jnp.float32)


def _t_mid(p_ref, g_ref, deg_ref, w_ref, b_ref, o_ref):
    dinv = _dinv_from(deg_ref)
    h = dinv * (p_ref[0] + p_ref[1] + g_ref[...]) + b_ref[...]
    h = jnp.maximum(h, 0.0)
    o_ref[...] = dinv * jnp.dot(h, w_ref[...],
                                preferred_element_type=jnp.float32)


def _t_last(p_ref, g_ref, deg_ref, b_ref, o_ref):
    dinv = _dinv_from(deg_ref)
    o_ref[...] = dinv * (p_ref[0] + p_ref[1] + g_ref[...]) + b_ref[...]


def _tc_call(body, grid, V, B, out_d, in_specs):
    return pl.pallas_call(
        body,
        grid=(grid,),
        in_specs=in_specs,
        out_specs=pl.BlockSpec((B, out_d), lambda i: (i, 0)),
        out_shape=jax.ShapeDtypeStruct((V, out_d), jnp.float32),
    )


def kernel(x, edge_index, W1, b1, W2, b2, W3, b3):
    N, DIN = x.shape
    DH = W1.shape[1]
    DOUT = W3.shape[1]
    V = ((N + 1 + 2047) // 2048) * 2048
    src = edge_index[0]
    dst = edge_index[1]
    E = src.shape[0]
    T = -(-E // (_NW * _C))
    EP = _NW * T * _C
    pad = EP - E
    src3 = jnp.concatenate([src, jnp.full((pad,), N, jnp.int32)]).reshape(_NW, T, _C)
    dst3 = jnp.concatenate([dst, jnp.full((pad,), N, jnp.int32)]).reshape(_NW, T, _C)

    x_p = jnp.pad(x, ((0, V - N), (0, 0)))
    W3p = jnp.pad(W3, ((0, 0), (0, 16 - DOUT)))
    b1r = b1.reshape(1, DH)
    b2r = b2.reshape(1, DH)
    b3r = jnp.pad(b3, (0, 16 - DOUT)).reshape(1, 16)

    ones16 = jnp.ones((_C, 16), jnp.float32)
    zeros16 = jnp.zeros((_C, 16), jnp.float32)
    zerosD = jnp.zeros((_C, DH), jnp.float32)

    deg_k = _make_deg_kernel(V, T)
    prop_k = _make_prop_kernel(V, DH, T)
    prop_k16 = _make_prop_kernel(V, 16, T)

    deg = deg_k(ones16, zeros16, dst3)

    B = 512
    G = V // B
    spec_w = pl.BlockSpec((DIN, DH), lambda i: (0, 0))
    spec_deg = pl.BlockSpec((2, B, 16), lambda i: (0, i, 0))
    spec_row = pl.BlockSpec((B, DH), lambda i: (i, 0))
    spec_row16 = pl.BlockSpec((B, 16), lambda i: (i, 0))
    spec_p = pl.BlockSpec((2, B, DH), lambda i: (0, i, 0))
    spec_p16 = pl.BlockSpec((2, B, 16), lambda i: (0, i, 0))
    spec_b = pl.BlockSpec((1, DH), lambda i: (0, 0))
    spec_b16 = pl.BlockSpec((1, 16), lambda i: (0, 0))

    g1 = _tc_call(_t_first, G, V, B, DH,
                  [spec_row, spec_w, spec_deg])(x_p, W1, deg)
    p1 = prop_k(g1, zerosD, src3, dst3)
    g2 = _tc_call(_t_mid, G, V, B, DH,
                  [spec_p, spec_row, spec_deg, spec_w, spec_b])(p1, g1, deg, W2, b1r)
    p2 = prop_k(g2, zerosD, src3, dst3)
    spec_w16 = pl.BlockSpec((DH, 16), lambda i: (0, 0))
    g3 = _tc_call(_t_mid, G, V, B, 16,
                  [spec_p, spec_row, spec_deg, spec_w16, spec_b])(p2, g2, deg, W3p, b2r)
    p3 = prop_k16(g3, zeros16, src3, dst3)
    out = _tc_call(_t_last, G, V, B, 16,
                   [spec_p16, spec_row16, spec_deg, spec_b16])(p3, g3, deg, b3r)
    return out[:N, :DOUT]
